# Initial kernel scaffold; baseline (speedup 1.0000x reference)
import jax
import jax.numpy as jnp
from jax.experimental import pallas as pl


def kernel(x, seq_len):
    raise NotImplementedError("write your pallas kernel here")


# speedup vs baseline: 2.8248x; 2.8248x over previous
import functools

import jax
import jax.numpy as jnp
from jax import lax
from jax.experimental import pallas as pl
from jax.experimental.pallas import tpu as pltpu
from jax.experimental.pallas import tpu_sc as plsc

MAX_PAD_LEN = 3072
MAX_SEQ_LEN = 2048
MIN_SEG_LEN = 19
MAX_SEG_LEN = 32
MAX_NUM_SEG = MAX_SEQ_LEN // MIN_SEG_LEN + 1
B, T, D = 16, 2048, 512
M = MAX_NUM_SEG * MAX_SEG_LEN * 2

NW = 32
RPW = B * MAX_PAD_LEN // NW
C = 64
NCHUNK = RPW // C
NV = D // 16


def _prep(seq_len):
    bm = B * MAX_NUM_SEG
    key = jax.random.key(42)
    k_scale, k_len = jax.random.split(key)
    indices = jnp.broadcast_to(
        jnp.arange(MAX_SEG_LEN * 2, dtype=jnp.float32)[None, :],
        (bm, MAX_SEG_LEN * 2))
    scales = jax.random.uniform(k_scale, (bm,), dtype=jnp.float32) + 0.5
    idx_scaled = indices / scales[:, None]
    idx_scaled_fl = jnp.floor(idx_scaled)
    lambda_ = idx_scaled - idx_scaled_fl
    len_seg = jax.random.randint(
        k_len, (bm, 1), MIN_SEG_LEN, MAX_SEG_LEN, dtype=jnp.int32)
    idx_mask = idx_scaled_fl < (len_seg - 1).astype(jnp.float32)
    offset = jnp.cumsum(len_seg.reshape(B, -1), axis=-1)
    offset = jnp.pad(offset[:, :-1], ((0, 0), (1, 0))).reshape(-1, 1)
    idx_scaled_org = idx_scaled_fl + offset.astype(jnp.float32)
    len_seq_rp = jnp.repeat(seq_len.astype(jnp.int32), MAX_NUM_SEG)
    idx_mask_org = idx_scaled_org < (len_seq_rp - 1).astype(jnp.float32)[:, None]
    mask_b = (idx_mask & idx_mask_org).reshape(B, M)
    ifl_b = idx_scaled_org.reshape(B, M).astype(jnp.int32)
    lam_b = lambda_.reshape(B, M)

    cum = jnp.cumsum(mask_b.astype(jnp.int32), axis=-1)
    count = jnp.minimum(cum[:, -1], MAX_PAD_LEN)
    p = jnp.arange(MAX_PAD_LEN, dtype=jnp.int32)
    jj = jax.vmap(lambda c: jnp.searchsorted(c, p + 1, side='left'))(cum)
    valid = p[None, :] < count[:, None]
    jjc = jnp.clip(jj, 0, M - 1)
    ifl = jnp.take_along_axis(ifl_b, jjc, axis=1)
    lam = jnp.take_along_axis(lam_b, jjc, axis=1)
    i0 = jnp.clip(ifl, 0, T - 2)
    g0 = jnp.arange(B, dtype=jnp.int32)[:, None] * T + i0
    w1 = jnp.where(valid, lam, 0.0).astype(jnp.float32)
    w0 = jnp.where(valid, 1.0 - lam, 0.0).astype(jnp.float32)
    w_ids = jnp.arange(NW, dtype=jnp.int32)
    nv = jnp.clip(count[w_ids * RPW // MAX_PAD_LEN] - (w_ids % (MAX_PAD_LEN // RPW)) * RPW,
                  0, RPW).astype(jnp.int32)
    return g0.reshape(-1), w0.reshape(-1), w1.reshape(-1), nv


def _sc_body(x_hbm, g0_hbm, g1_hbm, w0_hbm, w1_hbm, nv_hbm, out_hbm,
             g0_v, g1_v, w0_v, w1_v, a_v, b_v, z_v, nv_v, sem0, sem1):
    wid = lax.axis_index("s") * 2 + lax.axis_index("c")
    pltpu.sync_copy(nv_hbm, nv_v)
    cnt_splat = plsc.load_gather(nv_v, [lax.broadcast(wid, (16,))])
    myc = cnt_splat[0]

    zv = jnp.zeros((16,), jnp.float32)

    @pl.loop(0, C)
    def _zero(r):
        for v in range(NV):
            z_v[r, pl.ds(v * 16, 16)] = zv

    @pl.loop(0, NCHUNK)
    def _chunk(c):
        @pl.when(c * C < myc)
        def _gather_path():
            pltpu.sync_copy(g0_hbm.at[wid, c], g0_v)
            pltpu.sync_copy(g1_hbm.at[wid, c], g1_v)
            pltpu.sync_copy(w0_hbm.at[wid, c], w0_v)
            pltpu.sync_copy(w1_hbm.at[wid, c], w1_v)
            cp0 = pltpu.async_copy(x_hbm.at[g0_v], a_v, sem0)
            cp1 = pltpu.async_copy(x_hbm.at[g1_v], b_v, sem1)
            cp0.wait()
            cp1.wait()

            @pl.loop(0, C)
            def _row(r):
                w0s = w0_v[r]
                w1s = w1_v[r]
                for v in range(NV):
                    sl = pl.ds(v * 16, 16)
                    a_v[r, sl] = w0s * a_v[r, sl] + w1s * b_v[r, sl]

            pltpu.sync_copy(a_v, out_hbm.at[wid, c])

        @pl.when(c * C >= myc)
        def _zero_path():
            pltpu.sync_copy(z_v, out_hbm.at[wid, c])


def kernel(x, seq_len):
    g0, w0, w1, nv = _prep(seq_len)
    xf = x.reshape(B * T, D)
    g0_r = g0.reshape(NW, NCHUNK, C)
    g1_r = g0_r + 1
    w0_r = jnp.broadcast_to(w0.reshape(NW, NCHUNK, C, 1), (NW, NCHUNK, C, 16))
    w1_r = jnp.broadcast_to(w1.reshape(NW, NCHUNK, C, 1), (NW, NCHUNK, C, 16))

    mesh = plsc.VectorSubcoreMesh(core_axis_name="c", subcore_axis_name="s")
    run = functools.partial(
        pl.kernel,
        out_type=jax.ShapeDtypeStruct((NW, NCHUNK, C, D), jnp.float32),
        mesh=mesh,
        compiler_params=pltpu.CompilerParams(needs_layout_passes=False),
        scratch_types=[
            pltpu.VMEM((C,), jnp.int32),
            pltpu.VMEM((C,), jnp.int32),
            pltpu.VMEM((C, 16), jnp.float32),
            pltpu.VMEM((C, 16), jnp.float32),
            pltpu.VMEM((C, D), jnp.float32),
            pltpu.VMEM((C, D), jnp.float32),
            pltpu.VMEM((C, D), jnp.float32),
            pltpu.VMEM((NW,), jnp.int32),
            pltpu.SemaphoreType.DMA,
            pltpu.SemaphoreType.DMA,
        ],
    )(_sc_body)
    out = run(xf, g0_r, g1_r, w0_r.astype(jnp.float32), w1_r.astype(jnp.float32), nv)
    return out.reshape(B, MAX_PAD_LEN, D)


# speedup vs baseline: 4.5273x; 1.6027x over previous
import functools

import jax
import jax.numpy as jnp
from jax import lax
from jax.experimental import pallas as pl
from jax.experimental.pallas import tpu as pltpu
from jax.experimental.pallas import tpu_sc as plsc

MAX_PAD_LEN = 3072
MAX_SEQ_LEN = 2048
MIN_SEG_LEN = 19
MAX_SEG_LEN = 32
MAX_NUM_SEG = MAX_SEQ_LEN // MIN_SEG_LEN + 1
B, T, D = 16, 2048, 512
M = MAX_NUM_SEG * MAX_SEG_LEN * 2

NW = 32
C = 64
CHB = MAX_PAD_LEN // C
NCH = B * CHB // NW
NV = D // 16


def _prep(seq_len):
    bm = B * MAX_NUM_SEG
    key = jax.random.key(42)
    k_scale, k_len = jax.random.split(key)
    indices = jnp.broadcast_to(
        jnp.arange(MAX_SEG_LEN * 2, dtype=jnp.float32)[None, :],
        (bm, MAX_SEG_LEN * 2))
    scales = jax.random.uniform(k_scale, (bm,), dtype=jnp.float32) + 0.5
    idx_scaled = indices / scales[:, None]
    idx_scaled_fl = jnp.floor(idx_scaled)
    lambda_ = idx_scaled - idx_scaled_fl
    len_seg = jax.random.randint(
        k_len, (bm, 1), MIN_SEG_LEN, MAX_SEG_LEN, dtype=jnp.int32)
    idx_mask = idx_scaled_fl < (len_seg - 1).astype(jnp.float32)
    offset = jnp.cumsum(len_seg.reshape(B, -1), axis=-1)
    offset = jnp.pad(offset[:, :-1], ((0, 0), (1, 0))).reshape(-1, 1)
    idx_scaled_org = idx_scaled_fl + offset.astype(jnp.float32)
    len_seq_rp = jnp.repeat(seq_len.astype(jnp.int32), MAX_NUM_SEG)
    idx_mask_org = idx_scaled_org < (len_seq_rp - 1).astype(jnp.float32)[:, None]
    mask_b = (idx_mask & idx_mask_org).reshape(B, M)
    i0_b = jnp.clip(idx_scaled_org.reshape(B, M).astype(jnp.int32), 0, T - 2)
    lam_b = lambda_.reshape(B, M)

    cum = jnp.cumsum(mask_b.astype(jnp.int32), axis=-1)
    count = jnp.minimum(cum[:, -1], MAX_PAD_LEN)
    pos = cum - 1
    scat = jnp.where(mask_b & (pos < MAX_PAD_LEN), pos, MAX_PAD_LEN)
    vals = jnp.stack([i0_b.astype(jnp.float32), lam_b], axis=-1)
    inv = jnp.zeros((B, MAX_PAD_LEN + 1, 2), jnp.float32)
    b_ix = jnp.arange(B, dtype=jnp.int32)[:, None]
    inv = inv.at[b_ix, scat].set(vals, mode="drop")
    i0g = inv[:, :MAX_PAD_LEN, 0].astype(jnp.int32)
    lamg = inv[:, :MAX_PAD_LEN, 1]
    p = jnp.arange(MAX_PAD_LEN, dtype=jnp.int32)
    valid = p[None, :] < count[:, None]
    g0 = (b_ix * T + i0g).astype(jnp.float32)
    w1 = jnp.where(valid, lamg, 0.0)
    w0 = jnp.where(valid, 1.0 - lamg, 0.0)

    ngc = (count + C - 1) // C
    j = jnp.arange(CHB, dtype=jnp.int32)
    is_zero = (j[None, :] >= ngc[:, None]).reshape(-1)
    order = jnp.argsort(is_zero, stable=True).astype(jnp.int32)
    G = jnp.sum(ngc).astype(jnp.int32)
    slot_map = order.reshape(NCH, NW).T
    w_ids = jnp.arange(NW, dtype=jnp.int32)
    ng = jnp.maximum(0, (G - w_ids + NW - 1) // NW).astype(jnp.int32)
    gb = jnp.zeros((NW, NW), jnp.int32).at[:, :NCH].set(slot_map * C)

    packed = jnp.stack([g0, w0, w1], axis=-1).reshape(B * CHB, C, 3)
    packed = packed[slot_map.reshape(-1)].reshape(NW, NCH, C, 3)
    g0_r = packed[..., 0].astype(jnp.int32)
    w0_r = jnp.broadcast_to(packed[..., 1:2], (NW, NCH, C, 16))
    w1_r = jnp.broadcast_to(packed[..., 2:3], (NW, NCH, C, 16))
    return g0_r, w0_r, w1_r, ng, gb


def _splat0(ref, i):
    return plsc.load_gather(ref, [lax.broadcast(i, (16,))])[0]


def _sc_body(x_hbm, g0_hbm, w0_hbm, w1_hbm, ng_hbm, gb_hbm, out_hbm,
             g0_v, g1_v, w0_v, w1_v, a_v, b_v, z_v, ng_v, gb_v, sem0, sem1):
    wid = lax.axis_index("s") * 2 + lax.axis_index("c")
    pltpu.sync_copy(ng_hbm, ng_v)
    pltpu.sync_copy(gb_hbm.at[wid], gb_v)
    myg = _splat0(ng_v, wid)

    zv = jnp.zeros((16,), jnp.float32)

    @pl.loop(0, C)
    def _zero(r):
        for v in range(NV):
            z_v[r, pl.ds(v * 16, 16)] = zv

    @pl.loop(0, NCH)
    def _chunk(c):
        base = pl.multiple_of(_splat0(gb_v, c), C)

        @pl.when(c < myg)
        def _gather_path():
            pltpu.sync_copy(g0_hbm.at[wid, c], g0_v)
            pltpu.sync_copy(w0_hbm.at[wid, c], w0_v)
            pltpu.sync_copy(w1_hbm.at[wid, c], w1_v)
            for v in range(C // 16):
                sl = pl.ds(v * 16, 16)
                g1_v[sl] = g0_v[sl] + 1
            cp0 = pltpu.async_copy(x_hbm.at[g0_v], a_v, sem0)
            cp1 = pltpu.async_copy(x_hbm.at[g1_v], b_v, sem1)
            cp0.wait()
            cp1.wait()

            @pl.loop(0, C)
            def _row(r):
                w0s = w0_v[r]
                w1s = w1_v[r]
                for v in range(NV):
                    sl = pl.ds(v * 16, 16)
                    a_v[r, sl] = w0s * a_v[r, sl] + w1s * b_v[r, sl]

            pltpu.sync_copy(a_v, out_hbm.at[pl.ds(base, C)])

        @pl.when(c >= myg)
        def _zero_path():
            pltpu.sync_copy(z_v, out_hbm.at[pl.ds(base, C)])


def kernel(x, seq_len):
    g0_r, w0_r, w1_r, ng, gb = _prep(seq_len)
    xf = x.reshape(B * T, D)

    mesh = plsc.VectorSubcoreMesh(core_axis_name="c", subcore_axis_name="s")
    run = functools.partial(
        pl.kernel,
        out_type=jax.ShapeDtypeStruct((B * MAX_PAD_LEN, D), jnp.float32),
        mesh=mesh,
        compiler_params=pltpu.CompilerParams(needs_layout_passes=False),
        scratch_types=[
            pltpu.VMEM((C,), jnp.int32),
            pltpu.VMEM((C,), jnp.int32),
            pltpu.VMEM((C, 16), jnp.float32),
            pltpu.VMEM((C, 16), jnp.float32),
            pltpu.VMEM((C, D), jnp.float32),
            pltpu.VMEM((C, D), jnp.float32),
            pltpu.VMEM((C, D), jnp.float32),
            pltpu.VMEM((NW,), jnp.int32),
            pltpu.VMEM((NW,), jnp.int32),
            pltpu.SemaphoreType.DMA,
            pltpu.SemaphoreType.DMA,
        ],
    )(_sc_body)
    out = run(xf, g0_r, w0_r, w1_r, ng, gb)
    return out.reshape(B, MAX_PAD_LEN, D)


# speedup vs baseline: 5.3583x; 1.1836x over previous
import functools

import jax
import jax.numpy as jnp
import numpy as np
from jax import lax
from jax.experimental import pallas as pl
from jax.experimental.pallas import tpu as pltpu
from jax.experimental.pallas import tpu_sc as plsc

MAX_PAD_LEN = 3072
MAX_SEQ_LEN = 2048
MIN_SEG_LEN = 19
MAX_SEG_LEN = 32
MAX_NUM_SEG = MAX_SEQ_LEN // MIN_SEG_LEN + 1
B, T, D = 16, 2048, 512
M = MAX_NUM_SEG * MAX_SEG_LEN * 2

NW = 32
C = 64
CHB = MAX_PAD_LEN // C
NCHUNKS = B * CHB
NCH = NCHUNKS // NW
NV = D // 16


def _consts():
    def impl():
        bm = B * MAX_NUM_SEG
        key = jax.random.key(42)
        k_scale, k_len = jax.random.split(key)
        indices = jnp.broadcast_to(
            jnp.arange(MAX_SEG_LEN * 2, dtype=jnp.float32)[None, :],
            (bm, MAX_SEG_LEN * 2))
        scales = jax.random.uniform(k_scale, (bm,), dtype=jnp.float32) + 0.5
        idx_scaled = indices / scales[:, None]
        idx_scaled_fl = jnp.floor(idx_scaled)
        lambda_ = idx_scaled - idx_scaled_fl
        len_seg = jax.random.randint(
            k_len, (bm, 1), MIN_SEG_LEN, MAX_SEG_LEN, dtype=jnp.int32)
        idx_mask = idx_scaled_fl < (len_seg - 1).astype(jnp.float32)
        offset = jnp.cumsum(len_seg.reshape(B, -1), axis=-1)
        offset = jnp.pad(offset[:, :-1], ((0, 0), (1, 0))).reshape(-1, 1)
        idx_scaled_org = idx_scaled_fl + offset.astype(jnp.float32)
        i0 = jnp.clip(idx_scaled_org.astype(jnp.int32), 0, T - 2)
        vals = jnp.stack(
            [i0.astype(jnp.float32), lambda_], axis=-1).reshape(B, M, 2)
        return (idx_mask.reshape(B, M), idx_scaled_org.reshape(B, M), vals)

    cpu = jax.local_devices(backend="cpu")[:1]
    cpu_mesh = jax.make_mesh((1,), ("_c",), devices=cpu)
    with jax.set_mesh(cpu_mesh):
        out = jax.jit(impl)()
        return tuple(np.asarray(o) for o in out)


_IDX_MASK, _THR, _VALS = _consts()


def _prep(seq_len):
    mask_b = jnp.asarray(_IDX_MASK) & (
        jnp.asarray(_THR) < (seq_len.astype(jnp.float32) - 1.0)[:, None])
    cum = jnp.cumsum(mask_b.astype(jnp.int32), axis=-1)
    count = jnp.minimum(cum[:, -1], MAX_PAD_LEN)
    pos = cum - 1
    scat = jnp.where(mask_b & (pos < MAX_PAD_LEN), pos, MAX_PAD_LEN)
    inv = jnp.zeros((B, MAX_PAD_LEN + 1, 2), jnp.float32)
    b_ix = jnp.arange(B, dtype=jnp.int32)[:, None]
    inv = inv.at[b_ix, scat].set(jnp.asarray(_VALS), mode="drop")
    i0g = inv[:, :MAX_PAD_LEN, 0].astype(jnp.int32)
    lamg = inv[:, :MAX_PAD_LEN, 1]
    p = jnp.arange(MAX_PAD_LEN, dtype=jnp.int32)
    valid = p[None, :] < count[:, None]
    g0 = b_ix * T + i0g
    w1 = jnp.where(valid, lamg, 0.0)
    w0 = jnp.where(valid, 1.0 - lamg, 0.0)
    g0_r = g0.reshape(NCHUNKS, 1, C)
    w_r = jnp.concatenate(
        [w0.reshape(NCHUNKS, 1, C), w1.reshape(NCHUNKS, 1, C)], axis=-1)

    ngc = (count + C - 1) // C
    j = jnp.arange(CHB, dtype=jnp.int32)
    is_zero = (j[None, :] >= ngc[:, None]).reshape(-1)
    order = jnp.argsort(is_zero, stable=True).astype(jnp.int32)
    G = jnp.sum(ngc).astype(jnp.int32)
    slot_map = order.reshape(NCH, NW).T
    w_ids = jnp.arange(NW, dtype=jnp.int32)
    ng = jnp.maximum(0, (G - w_ids + NW - 1) // NW).astype(jnp.int32)
    gc = jnp.zeros((NW, NW), jnp.int32).at[:, :NCH].set(slot_map)
    return g0_r, w_r, ng, gc


def _splat0(ref, i):
    return plsc.load_gather(ref, [lax.broadcast(i, (16,))])[0]


def _splat_row(ref, i):
    z = lax.broadcast(jnp.int32(0), (16,))
    return plsc.load_gather(ref, [z, lax.broadcast(i, (16,))])


def _sc_body(x_hbm, g0_hbm, w_hbm, ng_hbm, gc_hbm, out_hbm,
             g0_v, i0_v, i1_v, w_v, a_v, b_v, z_v, ng_v, gc_v, sem0, sem1):
    wid = lax.axis_index("s") * 2 + lax.axis_index("c")
    pltpu.sync_copy(ng_hbm, ng_v)
    pltpu.sync_copy(gc_hbm.at[wid], gc_v)
    myg = _splat0(ng_v, wid)

    zv = jnp.zeros((16,), jnp.float32)

    @pl.loop(0, C)
    def _zero(r):
        for v in range(NV):
            z_v[r, pl.ds(v * 16, 16)] = zv

    @pl.loop(0, NCH)
    def _chunk(c):
        gcid = _splat0(gc_v, c)
        base = pl.multiple_of(gcid * C, C)

        @pl.when(c < myg)
        def _gather_path():
            pltpu.sync_copy(g0_hbm.at[gcid], g0_v)
            pltpu.sync_copy(w_hbm.at[gcid], w_v)
            for v in range(C // 16):
                sl = pl.ds(v * 16, 16)
                g = g0_v[0, sl]
                i0_v[sl] = g
                i1_v[sl] = g + 1
            cp0 = pltpu.async_copy(x_hbm.at[i0_v], a_v, sem0)
            cp1 = pltpu.async_copy(x_hbm.at[i1_v], b_v, sem1)
            cp0.wait()
            cp1.wait()

            @pl.loop(0, C)
            def _row(r):
                w0s = _splat_row(w_v, r)
                w1s = _splat_row(w_v, C + r)
                for v in range(NV):
                    sl = pl.ds(v * 16, 16)
                    a_v[r, sl] = w0s * a_v[r, sl] + w1s * b_v[r, sl]

            pltpu.sync_copy(a_v, out_hbm.at[pl.ds(base, C)])

        @pl.when(c >= myg)
        def _zero_path():
            pltpu.sync_copy(z_v, out_hbm.at[pl.ds(base, C)])


def kernel(x, seq_len):
    g0_r, w_r, ng, gc = _prep(seq_len)
    xf = x.reshape(B * T, D)

    mesh = plsc.VectorSubcoreMesh(core_axis_name="c", subcore_axis_name="s")
    run = functools.partial(
        pl.kernel,
        out_type=jax.ShapeDtypeStruct((B * MAX_PAD_LEN, D), jnp.float32),
        mesh=mesh,
        compiler_params=pltpu.CompilerParams(needs_layout_passes=False),
        scratch_types=[
            pltpu.VMEM((1, C), jnp.int32),
            pltpu.VMEM((C,), jnp.int32),
            pltpu.VMEM((C,), jnp.int32),
            pltpu.VMEM((1, 2 * C), jnp.float32),
            pltpu.VMEM((C, D), jnp.float32),
            pltpu.VMEM((C, D), jnp.float32),
            pltpu.VMEM((C, D), jnp.float32),
            pltpu.VMEM((NW,), jnp.int32),
            pltpu.VMEM((NW,), jnp.int32),
            pltpu.SemaphoreType.DMA,
            pltpu.SemaphoreType.DMA,
        ],
    )(_sc_body)
    out = run(xf, g0_r, w_r, ng, gc)
    return out.reshape(B, MAX_PAD_LEN, D)


# speedup vs baseline: 13.0994x; 2.4447x over previous
import functools

import jax
import jax.numpy as jnp
import numpy as np
from jax import lax
from jax.experimental import pallas as pl
from jax.experimental.pallas import tpu as pltpu
from jax.experimental.pallas import tpu_sc as plsc

MAX_PAD_LEN = 3072
MAX_SEQ_LEN = 2048
MIN_SEG_LEN = 19
MAX_SEG_LEN = 32
MAX_NUM_SEG = MAX_SEQ_LEN // MIN_SEG_LEN + 1
SEG_W = MAX_SEG_LEN * 2
B, T, D = 16, 2048, 512
M = MAX_NUM_SEG * SEG_W

NW = 32
C = 64
CHB = MAX_PAD_LEN // C
NCHUNKS = B * CHB
NCH = NCHUNKS // NW
NV = D // 16
PACK_SHIFT = 13


def _consts():
    def impl():
        bm = B * MAX_NUM_SEG
        key = jax.random.key(42)
        k_scale, k_len = jax.random.split(key)
        indices = jnp.broadcast_to(
            jnp.arange(SEG_W, dtype=jnp.float32)[None, :], (bm, SEG_W))
        scales = jax.random.uniform(k_scale, (bm,), dtype=jnp.float32) + 0.5
        idx_scaled = indices / scales[:, None]
        idx_scaled_fl = jnp.floor(idx_scaled)
        lambda_ = idx_scaled - idx_scaled_fl
        len_seg = jax.random.randint(
            k_len, (bm, 1), MIN_SEG_LEN, MAX_SEG_LEN, dtype=jnp.int32)
        offset = jnp.cumsum(len_seg.reshape(B, -1), axis=-1)
        offset = jnp.pad(offset[:, :-1], ((0, 0), (1, 0)))
        idx_scaled_org = idx_scaled_fl.reshape(B, MAX_NUM_SEG, SEG_W) + \
            offset.astype(jnp.float32)[:, :, None]
        i0 = jnp.clip(idx_scaled_org.astype(jnp.int32), 0, T - 2)
        g0f = (jnp.arange(B, dtype=jnp.int32)[:, None, None] * T
               + i0).astype(jnp.float32)
        meta = jnp.concatenate(
            [g0f.reshape(B * MAX_NUM_SEG, SEG_W),
             lambda_.reshape(B * MAX_NUM_SEG, SEG_W)],
            axis=-1)
        fl = idx_scaled_fl.reshape(B, MAX_NUM_SEG, SEG_W)
        len1 = (len_seg - 1).reshape(B, MAX_NUM_SEG)
        return fl, len1, offset, meta

    cpu = jax.local_devices(backend="cpu")[:1]
    cpu_mesh = jax.make_mesh((1,), ("_c",), devices=cpu)
    with jax.set_mesh(cpu_mesh):
        out = jax.jit(impl)()
        return tuple(np.asarray(o) for o in out)


_FL, _LEN1, _OFF, _META = _consts()


def _prep(seq_len):
    thr = jnp.minimum(
        jnp.asarray(_LEN1, jnp.float32),
        (seq_len[:, None] - 1 - jnp.asarray(_OFF)).astype(jnp.float32))
    v = jnp.sum(jnp.asarray(_FL) < thr[:, :, None], axis=-1,
                dtype=jnp.int32)
    cums = jnp.cumsum(v, axis=-1)
    seg_start = cums - v
    count = jnp.minimum(cums[:, -1], MAX_PAD_LEN)
    seg_ids = jnp.arange(MAX_NUM_SEG, dtype=jnp.int32)
    pack = (seg_ids << PACK_SHIFT) + seg_start
    p = jnp.arange(MAX_PAD_LEN, dtype=jnp.int32)
    le = seg_start[:, None, :] <= p[None, :, None]
    pmax = jnp.max(jnp.where(le, pack[:, None, :], 0), axis=-1)
    s_p = pmax >> PACK_SHIFT
    start_p = pmax & ((1 << PACK_SHIFT) - 1)
    jj = s_p * SEG_W + (p[None, :] - start_p)
    b_ix = jnp.arange(B, dtype=jnp.int32)[:, None]
    valid = p[None, :] < count[:, None]
    jj_enc = jnp.where(valid, b_ix * M + jj, -1)
    jj_r = jj_enc.reshape(NCHUNKS, 1, C)

    ngc = (count + C - 1) // C
    j = jnp.arange(CHB, dtype=jnp.int32)
    is_zero = (j[None, :] >= ngc[:, None]).reshape(-1)
    order = jnp.argsort(is_zero, stable=True).astype(jnp.int32)
    G = jnp.sum(ngc).astype(jnp.int32)
    slot_map = order.reshape(NCH, NW).T
    w_ids = jnp.arange(NW, dtype=jnp.int32)
    ng = jnp.maximum(0, (G - w_ids + NW - 1) // NW).astype(jnp.int32)
    gc = jnp.zeros((NW, NW), jnp.int32).at[:, :NCH].set(slot_map)
    return jj_r, ng, gc


def _splat0(ref, i):
    return plsc.load_gather(ref, [lax.broadcast(i, (16,))])[0]


def _sc_body(x_hbm, meta_hbm, jj_hbm, ng_hbm, gc_hbm, out_hbm,
             jje_v, jjc_v, sg_v, meta_v, i0_v, i1_v, w0_v, w1_v,
             a_v, b_v, z_v, ng_v, gc_v, sem0, sem1, semz):
    wid = lax.axis_index("s") * 2 + lax.axis_index("c")
    pltpu.sync_copy(ng_hbm, ng_v)
    pltpu.sync_copy(gc_hbm.at[wid], gc_v)
    myg = _splat0(ng_v, wid)

    zv = jnp.zeros((16,), jnp.float32)
    lanes = lax.iota(jnp.int32, 16)

    @pl.loop(0, C)
    def _zero(r):
        for vv in range(NV):
            z_v[r, pl.ds(vv * 16, 16)] = zv

    @pl.loop(0, NCH)
    def _chunk(c):
        gcid = _splat0(gc_v, c)
        base = pl.multiple_of(gcid * C, C)

        @pl.when(c < myg)
        def _gather_path():
            pltpu.sync_copy(jj_hbm.at[gcid], jje_v)
            for k in range(C // 16):
                sl = pl.ds(k * 16, 16)
                jc = jnp.maximum(jje_v[0, sl], 0)
                jjc_v[sl] = jc
                sg_v[sl] = jc >> 6
            pltpu.async_copy(meta_hbm.at[sg_v], meta_v, sem0).wait()
            for k in range(C // 16):
                sl = pl.ds(k * 16, 16)
                row = lax.broadcast(jnp.int32(k * 16), (16,)) + lanes
                col = jjc_v[sl] & (SEG_W - 1)
                g0f = plsc.load_gather(meta_v, [row, col])
                lam = plsc.load_gather(meta_v, [row, col + SEG_W])
                mf = jnp.where(jje_v[0, sl] >= 0, 1.0, 0.0)
                g0i = g0f.astype(jnp.int32)
                i0_v[sl] = g0i
                i1_v[sl] = g0i + 1
                w1f = lam * mf
                w0_v[sl] = mf - w1f
                w1_v[sl] = w1f
            cp0 = pltpu.async_copy(x_hbm.at[i0_v], a_v, sem0)
            cp1 = pltpu.async_copy(x_hbm.at[i1_v], b_v, sem1)
            cp0.wait()
            cp1.wait()

            @pl.loop(0, C)
            def _row(r):
                w0s = plsc.load_gather(w0_v, [lax.broadcast(r, (16,))])
                w1s = plsc.load_gather(w1_v, [lax.broadcast(r, (16,))])
                for vv in range(NV):
                    sl = pl.ds(vv * 16, 16)
                    a_v[r, sl] = w0s * a_v[r, sl] + w1s * b_v[r, sl]

            pltpu.sync_copy(a_v, out_hbm.at[pl.ds(base, C)])

        @pl.when(c >= myg)
        def _zero_path():
            pltpu.async_copy(z_v, out_hbm.at[pl.ds(base, C)], semz).wait()


def kernel(x, seq_len):
    jj_r, ng, gc = _prep(seq_len)
    xf = x.reshape(B * T, D)
    meta = jnp.asarray(_META)

    mesh = plsc.VectorSubcoreMesh(core_axis_name="c", subcore_axis_name="s")
    run = functools.partial(
        pl.kernel,
        out_type=jax.ShapeDtypeStruct((B * MAX_PAD_LEN, D), jnp.float32),
        mesh=mesh,
        compiler_params=pltpu.CompilerParams(needs_layout_passes=False),
        scratch_types=[
            pltpu.VMEM((1, C), jnp.int32),
            pltpu.VMEM((C,), jnp.int32),
            pltpu.VMEM((C,), jnp.int32),
            pltpu.VMEM((C, 2 * SEG_W), jnp.float32),
            pltpu.VMEM((C,), jnp.int32),
            pltpu.VMEM((C,), jnp.int32),
            pltpu.VMEM((C,), jnp.float32),
            pltpu.VMEM((C,), jnp.float32),
            pltpu.VMEM((C, D), jnp.float32),
            pltpu.VMEM((C, D), jnp.float32),
            pltpu.VMEM((C, D), jnp.float32),
            pltpu.VMEM((NW,), jnp.int32),
            pltpu.VMEM((NW,), jnp.int32),
            pltpu.SemaphoreType.DMA,
            pltpu.SemaphoreType.DMA,
            pltpu.SemaphoreType.DMA,
        ],
    )(_sc_body)
    out = run(xf, meta, jj_r, ng, gc)
    return out.reshape(B, MAX_PAD_LEN, D)


# speedup vs baseline: 17.3626x; 1.3255x over previous
import functools

import jax
import jax.numpy as jnp
import numpy as np
from jax import lax
from jax.experimental import pallas as pl
from jax.experimental.pallas import tpu as pltpu
from jax.experimental.pallas import tpu_sc as plsc

MAX_PAD_LEN = 3072
MAX_SEQ_LEN = 2048
MIN_SEG_LEN = 19
MAX_SEG_LEN = 32
MAX_NUM_SEG = MAX_SEQ_LEN // MIN_SEG_LEN + 1
SEG_W = MAX_SEG_LEN * 2
B, T, D = 16, 2048, 512
M = MAX_NUM_SEG * SEG_W

NW = 32
C = 32
CHB = MAX_PAD_LEN // C
NCHUNKS = B * CHB
NCH = NCHUNKS // NW
NPAIR = NCH // 2 + 1
NV = D // 16
PACK_SHIFT = 13
JJP = 128


def _consts():
    def impl():
        bm = B * MAX_NUM_SEG
        key = jax.random.key(42)
        k_scale, k_len = jax.random.split(key)
        indices = jnp.broadcast_to(
            jnp.arange(SEG_W, dtype=jnp.float32)[None, :], (bm, SEG_W))
        scales = jax.random.uniform(k_scale, (bm,), dtype=jnp.float32) + 0.5
        idx_scaled = indices / scales[:, None]
        idx_scaled_fl = jnp.floor(idx_scaled)
        lambda_ = idx_scaled - idx_scaled_fl
        len_seg = jax.random.randint(
            k_len, (bm, 1), MIN_SEG_LEN, MAX_SEG_LEN, dtype=jnp.int32)
        offset = jnp.cumsum(len_seg.reshape(B, -1), axis=-1)
        offset = jnp.pad(offset[:, :-1], ((0, 0), (1, 0)))
        idx_scaled_org = idx_scaled_fl.reshape(B, MAX_NUM_SEG, SEG_W) + \
            offset.astype(jnp.float32)[:, :, None]
        i0 = jnp.clip(idx_scaled_org.astype(jnp.int32), 0, T - 2)
        g0f = (jnp.arange(B, dtype=jnp.int32)[:, None, None] * T
               + i0).astype(jnp.float32)
        meta = jnp.concatenate(
            [g0f.reshape(B * MAX_NUM_SEG, SEG_W),
             lambda_.reshape(B * MAX_NUM_SEG, SEG_W)],
            axis=-1)
        fl = idx_scaled_fl.reshape(B, MAX_NUM_SEG, SEG_W)
        len1 = (len_seg - 1).reshape(B, MAX_NUM_SEG)
        return fl, len1, offset, meta

    cpu = jax.local_devices(backend="cpu")[:1]
    cpu_mesh = jax.make_mesh((1,), ("_c",), devices=cpu)
    with jax.set_mesh(cpu_mesh):
        out = jax.jit(impl)()
        return tuple(np.asarray(o) for o in out)


_FL, _LEN1, _OFF, _META = _consts()


def _prep(seq_len):
    thr = jnp.minimum(
        jnp.asarray(_LEN1, jnp.float32),
        (seq_len[:, None] - 1 - jnp.asarray(_OFF)).astype(jnp.float32))
    v = jnp.sum(jnp.asarray(_FL) < thr[:, :, None], axis=-1,
                dtype=jnp.int32)
    cums = jnp.cumsum(v, axis=-1)
    seg_start = cums - v
    count = jnp.minimum(cums[:, -1], MAX_PAD_LEN)
    seg_ids = jnp.arange(MAX_NUM_SEG, dtype=jnp.int32)
    pack = (seg_ids << PACK_SHIFT) + seg_start
    p = jnp.arange(MAX_PAD_LEN, dtype=jnp.int32)
    le = seg_start[:, None, :] <= p[None, :, None]
    pmax = jnp.max(jnp.where(le, pack[:, None, :], 0), axis=-1)
    s_p = pmax >> PACK_SHIFT
    start_p = pmax & ((1 << PACK_SHIFT) - 1)
    jj = s_p * SEG_W + (p[None, :] - start_p)
    b_ix = jnp.arange(B, dtype=jnp.int32)[:, None]
    valid = p[None, :] < count[:, None]
    jj_enc = jnp.where(valid, b_ix * M + jj, -1)
    jj_r = jnp.pad(jj_enc.reshape(NCHUNKS, C),
                   ((0, 0), (0, JJP - C)))

    ngc = (count + C - 1) // C
    j = jnp.arange(CHB, dtype=jnp.int32)
    is_zero = (j[None, :] >= ngc[:, None]).reshape(-1)
    order = jnp.argsort(is_zero, stable=True).astype(jnp.int32)
    G = jnp.sum(ngc).astype(jnp.int32)
    slot_map = order.reshape(NCH, NW).T
    w_ids = jnp.arange(NW, dtype=jnp.int32)
    ng = jnp.maximum(0, (G - w_ids + NW - 1) // NW).astype(jnp.int32)
    return jj_r, ng, slot_map


def _splat0(ref, i):
    return plsc.load_gather(ref, [lax.broadcast(i, (16,))])[0]


def _sc_body(x_hbm, meta_hbm, jj_hbm, ng_hbm, gc_hbm, out_hbm,
             jja_v, gc_v, ng_v, sg, meta, i0, i1, w0, w1, av, bv, ov,
             msem, xsem, ssem, semz):
    wid = lax.axis_index("s") * 2 + lax.axis_index("c")
    pltpu.sync_copy(ng_hbm, ng_v)
    pltpu.sync_copy(gc_hbm.at[wid], gc_v)
    myg = _splat0(ng_v, wid)
    lanes = lax.iota(jnp.int32, 16)

    pltpu.async_copy(jj_hbm.at[gc_v], jja_v, xsem[0]).wait()

    def build_sg(c, d):
        for k in range(C // 16):
            sl = pl.ds(k * 16, 16)
            sg[d][sl] = jnp.maximum(jja_v[c, sl], 0) >> 6

    def fire_meta(c, d):
        build_sg(c, d)
        pltpu.async_copy(meta_hbm.at[sg[d]], meta[d], msem[d])

    def consume_meta_fire_x(c, d):
        pltpu.make_async_copy(meta_hbm.at[pl.ds(0, C)], meta[d], msem[d]).wait()
        for k in range(C // 16):
            sl = pl.ds(k * 16, 16)
            je = jja_v[c, sl]
            jc = jnp.maximum(je, 0)
            row = lax.broadcast(jnp.int32(k * 16), (16,)) + lanes
            col = jc & (SEG_W - 1)
            g0f = plsc.load_gather(meta[d], [row, col])
            lam = plsc.load_gather(meta[d], [row, col + SEG_W])
            mf = jnp.where(je >= 0, 1.0, 0.0)
            g0i = g0f.astype(jnp.int32)
            i0[d][sl] = g0i
            i1[d][sl] = g0i + 1
            w1f = lam * mf
            w0[d][sl] = mf - w1f
            w1[d][sl] = w1f

        @pl.when(c + 2 < myg)
        def _():
            fire_meta(c + 2, d)

        pltpu.async_copy(x_hbm.at[i0[d]], av[d], xsem[d])
        pltpu.async_copy(x_hbm.at[i1[d]], bv[d], xsem[d])

    def compute_store(c, d):
        pltpu.make_async_copy(x_hbm.at[pl.ds(0, C)], av[d], xsem[d]).wait()
        pltpu.make_async_copy(x_hbm.at[pl.ds(0, C)], bv[d], xsem[d]).wait()

        @pl.when(c >= 2)
        def _():
            pltpu.make_async_copy(out_hbm.at[pl.ds(0, C)], ov[d], ssem[d]).wait()

        @pl.loop(0, C)
        def _row(r):
            w0s = plsc.load_gather(w0[d], [lax.broadcast(r, (16,))])
            w1s = plsc.load_gather(w1[d], [lax.broadcast(r, (16,))])
            for vv in range(NV):
                sl = pl.ds(vv * 16, 16)
                ov[d][r, sl] = w0s * av[d][r, sl] + w1s * bv[d][r, sl]

        base = pl.multiple_of(_splat0(gc_v, c) * C, C)
        pltpu.async_copy(ov[d], out_hbm.at[pl.ds(base, C)], ssem[d])

    @pl.when(myg >= 1)
    def _():
        fire_meta(0, 0)

    @pl.when(myg >= 2)
    def _():
        fire_meta(1, 1)

    @pl.loop(0, NPAIR)
    def _pair(i):
        e = 2 * i
        q = 2 * i + 1
        po = 2 * i - 1

        @pl.when(e < myg)
        def _():
            consume_meta_fire_x(e, 0)

        @pl.when((po >= 0) & (po < myg))
        def _():
            compute_store(po, 1)

        @pl.when(q < myg)
        def _():
            consume_meta_fire_x(q, 1)

        @pl.when(e < myg)
        def _():
            compute_store(e, 0)

    @pl.when(myg >= 1)
    def _():
        pltpu.make_async_copy(out_hbm.at[pl.ds(0, C)], ov[0], ssem[0]).wait()

    @pl.when(myg >= 2)
    def _():
        pltpu.make_async_copy(out_hbm.at[pl.ds(0, C)], ov[1], ssem[1]).wait()

    zv = jnp.zeros((16,), jnp.float32)

    @pl.loop(0, C)
    def _zero(r):
        for vv in range(NV):
            ov[0][r, pl.ds(vv * 16, 16)] = zv

    @pl.loop(myg, NCH)
    def _zchunk(c):
        base = pl.multiple_of(_splat0(gc_v, c) * C, C)
        pltpu.async_copy(ov[0], out_hbm.at[pl.ds(base, C)], semz).wait()


def kernel(x, seq_len):
    jj_r, ng, gc = _prep(seq_len)
    xf = x.reshape(B * T, D)
    meta = jnp.asarray(_META)

    mesh = plsc.VectorSubcoreMesh(core_axis_name="c", subcore_axis_name="s")
    run = functools.partial(
        pl.kernel,
        out_type=jax.ShapeDtypeStruct((B * MAX_PAD_LEN, D), jnp.float32),
        mesh=mesh,
        compiler_params=pltpu.CompilerParams(needs_layout_passes=False),
        scratch_types=[
            pltpu.VMEM((NCH, JJP), jnp.int32),
            pltpu.VMEM((NCH,), jnp.int32),
            pltpu.VMEM((NW,), jnp.int32),
            [pltpu.VMEM((C,), jnp.int32)] * 2,
            [pltpu.VMEM((C, 2 * SEG_W), jnp.float32)] * 2,
            [pltpu.VMEM((C,), jnp.int32)] * 2,
            [pltpu.VMEM((C,), jnp.int32)] * 2,
            [pltpu.VMEM((C,), jnp.float32)] * 2,
            [pltpu.VMEM((C,), jnp.float32)] * 2,
            [pltpu.VMEM((C, D), jnp.float32)] * 2,
            [pltpu.VMEM((C, D), jnp.float32)] * 2,
            [pltpu.VMEM((C, D), jnp.float32)] * 2,
            [pltpu.SemaphoreType.DMA] * 2,
            [pltpu.SemaphoreType.DMA] * 2,
            [pltpu.SemaphoreType.DMA] * 2,
            pltpu.SemaphoreType.DMA,
        ],
    )(_sc_body)
    out = run(xf, meta, jj_r, ng, gc)
    return out.reshape(B, MAX_PAD_LEN, D)


# speedup vs baseline: 17.7009x; 1.0195x over previous
import functools

import jax
import jax.numpy as jnp
import numpy as np
from jax import lax
from jax.experimental import pallas as pl
from jax.experimental.pallas import tpu as pltpu
from jax.experimental.pallas import tpu_sc as plsc

MAX_PAD_LEN = 3072
MAX_SEQ_LEN = 2048
MIN_SEG_LEN = 19
MAX_SEG_LEN = 32
MAX_NUM_SEG = MAX_SEQ_LEN // MIN_SEG_LEN + 1
SEG_W = MAX_SEG_LEN * 2
B, T, D = 16, 2048, 512
M = MAX_NUM_SEG * SEG_W

NW = 32
C = 32
CHB = MAX_PAD_LEN // C
NCHUNKS = B * CHB
NCH = NCHUNKS // NW
NPAIR = NCH // 2 + 1
NV = D // 16
PACK_SHIFT = 13
JJP = 128


def _consts():
    def impl():
        bm = B * MAX_NUM_SEG
        key = jax.random.key(42)
        k_scale, k_len = jax.random.split(key)
        indices = jnp.broadcast_to(
            jnp.arange(SEG_W, dtype=jnp.float32)[None, :], (bm, SEG_W))
        scales = jax.random.uniform(k_scale, (bm,), dtype=jnp.float32) + 0.5
        idx_scaled = indices / scales[:, None]
        idx_scaled_fl = jnp.floor(idx_scaled)
        lambda_ = idx_scaled - idx_scaled_fl
        len_seg = jax.random.randint(
            k_len, (bm, 1), MIN_SEG_LEN, MAX_SEG_LEN, dtype=jnp.int32)
        offset = jnp.cumsum(len_seg.reshape(B, -1), axis=-1)
        offset = jnp.pad(offset[:, :-1], ((0, 0), (1, 0)))
        idx_scaled_org = idx_scaled_fl.reshape(B, MAX_NUM_SEG, SEG_W) + \
            offset.astype(jnp.float32)[:, :, None]
        i0 = jnp.clip(idx_scaled_org.astype(jnp.int32), 0, T - 2)
        g0f = (jnp.arange(B, dtype=jnp.int32)[:, None, None] * T
               + i0).astype(jnp.float32)
        meta = jnp.concatenate(
            [g0f.reshape(B * MAX_NUM_SEG, SEG_W),
             lambda_.reshape(B * MAX_NUM_SEG, SEG_W)],
            axis=-1)
        fl = idx_scaled_fl.reshape(B, MAX_NUM_SEG, SEG_W)
        len1 = (len_seg - 1).reshape(B, MAX_NUM_SEG)
        return fl, len1, offset, meta

    cpu = jax.local_devices(backend="cpu")[:1]
    cpu_mesh = jax.make_mesh((1,), ("_c",), devices=cpu)
    with jax.set_mesh(cpu_mesh):
        out = jax.jit(impl)()
        return tuple(np.asarray(o) for o in out)


_FL, _LEN1, _OFF, _META = _consts()


def _prep(seq_len):
    thr = jnp.minimum(
        jnp.asarray(_LEN1, jnp.float32),
        (seq_len[:, None] - 1 - jnp.asarray(_OFF)).astype(jnp.float32))
    v = jnp.sum(jnp.asarray(_FL) < thr[:, :, None], axis=-1,
                dtype=jnp.int32)
    cums = jnp.cumsum(v, axis=-1)
    seg_start = cums - v
    count = jnp.minimum(cums[:, -1], MAX_PAD_LEN)
    seg_ids = jnp.arange(MAX_NUM_SEG, dtype=jnp.int32)
    pack = (seg_ids << PACK_SHIFT) + seg_start
    p = jnp.arange(MAX_PAD_LEN, dtype=jnp.int32)
    le = seg_start[:, None, :] <= p[None, :, None]
    pmax = jnp.max(jnp.where(le, pack[:, None, :], 0), axis=-1)
    s_p = pmax >> PACK_SHIFT
    start_p = pmax & ((1 << PACK_SHIFT) - 1)
    jj = s_p * SEG_W + (p[None, :] - start_p)
    b_ix = jnp.arange(B, dtype=jnp.int32)[:, None]
    valid = p[None, :] < count[:, None]
    jj_enc = jnp.where(valid, b_ix * M + jj, -1)
    jj_r = jnp.pad(jj_enc.reshape(NCHUNKS, C),
                   ((0, 0), (0, JJP - C)))

    ngc = (count + C - 1) // C
    j = jnp.arange(CHB, dtype=jnp.int32)
    is_zero = (j[None, :] >= ngc[:, None]).reshape(-1)
    order = jnp.argsort(is_zero, stable=True).astype(jnp.int32)
    G = jnp.sum(ngc).astype(jnp.int32)
    slot_map = order.reshape(NCH, NW).T
    w_ids = jnp.arange(NW, dtype=jnp.int32)
    ng = jnp.maximum(0, (G - w_ids + NW - 1) // NW).astype(jnp.int32)
    return jj_r, ng, slot_map


def _splat0(ref, i):
    return plsc.load_gather(ref, [lax.broadcast(i, (16,))])[0]


def _sc_body(x_hbm, meta_hbm, jj_hbm, ng_hbm, gc_hbm, out_hbm,
             jja_v, gc_v, ng_v, sg, meta, i0, i1, w0, w1, av, bv, ov, z_v,
             msem, xsem, ssem, semz):
    wid = lax.axis_index("s") * 2 + lax.axis_index("c")
    pltpu.sync_copy(ng_hbm, ng_v)
    pltpu.sync_copy(gc_hbm.at[wid], gc_v)
    myg = _splat0(ng_v, wid)
    lanes = lax.iota(jnp.int32, 16)
    zv = jnp.zeros((16,), jnp.float32)

    @pl.loop(0, C)
    def _zfill(r):
        for vv in range(NV):
            z_v[r, pl.ds(vv * 16, 16)] = zv

    pltpu.async_copy(jj_hbm.at[gc_v], jja_v, xsem[0]).wait()

    def build_sg(c, d):
        for k in range(C // 16):
            sl = pl.ds(k * 16, 16)
            sg[d][sl] = jnp.maximum(jja_v[c, sl], 0) >> 6

    def fire_meta(c, d):
        build_sg(c, d)
        pltpu.async_copy(meta_hbm.at[sg[d]], meta[d], msem[d])

    def consume_meta_fire_x(c, d):
        pltpu.make_async_copy(meta_hbm.at[pl.ds(0, C)], meta[d], msem[d]).wait()
        for k in range(C // 16):
            sl = pl.ds(k * 16, 16)
            je = jja_v[c, sl]
            jc = jnp.maximum(je, 0)
            row = lax.broadcast(jnp.int32(k * 16), (16,)) + lanes
            col = jc & (SEG_W - 1)
            g0f = plsc.load_gather(meta[d], [row, col])
            lam = plsc.load_gather(meta[d], [row, col + SEG_W])
            mf = jnp.where(je >= 0, 1.0, 0.0)
            g0i = g0f.astype(jnp.int32)
            i0[d][sl] = g0i
            i1[d][sl] = g0i + 1
            w1f = lam * mf
            w0[d][sl] = mf - w1f
            w1[d][sl] = w1f

        @pl.when(c + 2 < myg)
        def _():
            fire_meta(c + 2, d)

        pltpu.async_copy(x_hbm.at[i0[d]], av[d], xsem[d])
        pltpu.async_copy(x_hbm.at[i1[d]], bv[d], xsem[d])

    def compute_store(c, d):
        pltpu.make_async_copy(x_hbm.at[pl.ds(0, C)], av[d], xsem[d]).wait()
        pltpu.make_async_copy(x_hbm.at[pl.ds(0, C)], bv[d], xsem[d]).wait()

        @pl.when(c >= 2)
        def _():
            pltpu.make_async_copy(out_hbm.at[pl.ds(0, C)], ov[d], ssem[d]).wait()

        @pl.loop(0, C)
        def _row(r):
            w0s = plsc.load_gather(w0[d], [lax.broadcast(r, (16,))])
            w1s = plsc.load_gather(w1[d], [lax.broadcast(r, (16,))])
            for vv in range(NV):
                sl = pl.ds(vv * 16, 16)
                ov[d][r, sl] = w0s * av[d][r, sl] + w1s * bv[d][r, sl]

        base = pl.multiple_of(_splat0(gc_v, c) * C, C)
        pltpu.async_copy(ov[d], out_hbm.at[pl.ds(base, C)], ssem[d])

    @pl.when(myg >= 1)
    def _():
        fire_meta(0, 0)

    @pl.when(myg >= 2)
    def _():
        fire_meta(1, 1)

    @pl.loop(0, NPAIR)
    def _pair(i):
        e = 2 * i
        q = 2 * i + 1
        po = 2 * i - 1

        @pl.when(e < myg)
        def _():
            consume_meta_fire_x(e, 0)

        @pl.when((po >= 0) & (po < myg))
        def _():
            compute_store(po, 1)

        @pl.when(q < myg)
        def _():
            consume_meta_fire_x(q, 1)

        @pl.when(e < myg)
        def _():
            compute_store(e, 0)

        for zc in (myg + 2 * i, myg + 2 * i + 1):
            @pl.when(zc < NCH)
            def _(zc=zc):
                zb = pl.multiple_of(_splat0(gc_v, zc) * C, C)
                pltpu.async_copy(z_v, out_hbm.at[pl.ds(zb, C)], semz)

    @pl.when(myg >= 1)
    def _():
        pltpu.make_async_copy(out_hbm.at[pl.ds(0, C)], ov[0], ssem[0]).wait()

    @pl.when(myg >= 2)
    def _():
        pltpu.make_async_copy(out_hbm.at[pl.ds(0, C)], ov[1], ssem[1]).wait()

    @pl.loop(myg, NCH)
    def _zdrain(c):
        pltpu.make_async_copy(out_hbm.at[pl.ds(0, C)], z_v, semz).wait()


def kernel(x, seq_len):
    jj_r, ng, gc = _prep(seq_len)
    xf = x.reshape(B * T, D)
    meta = jnp.asarray(_META)

    mesh = plsc.VectorSubcoreMesh(core_axis_name="c", subcore_axis_name="s")
    run = functools.partial(
        pl.kernel,
        out_type=jax.ShapeDtypeStruct((B * MAX_PAD_LEN, D), jnp.float32),
        mesh=mesh,
        compiler_params=pltpu.CompilerParams(needs_layout_passes=False),
        scratch_types=[
            pltpu.VMEM((NCH, JJP), jnp.int32),
            pltpu.VMEM((NCH,), jnp.int32),
            pltpu.VMEM((NW,), jnp.int32),
            [pltpu.VMEM((C,), jnp.int32)] * 2,
            [pltpu.VMEM((C, 2 * SEG_W), jnp.float32)] * 2,
            [pltpu.VMEM((C,), jnp.int32)] * 2,
            [pltpu.VMEM((C,), jnp.int32)] * 2,
            [pltpu.VMEM((C,), jnp.float32)] * 2,
            [pltpu.VMEM((C,), jnp.float32)] * 2,
            [pltpu.VMEM((C, D), jnp.float32)] * 2,
            [pltpu.VMEM((C, D), jnp.float32)] * 2,
            [pltpu.VMEM((C, D), jnp.float32)] * 2,
            pltpu.VMEM((C, D), jnp.float32),
            [pltpu.SemaphoreType.DMA] * 2,
            [pltpu.SemaphoreType.DMA] * 2,
            [pltpu.SemaphoreType.DMA] * 2,
            pltpu.SemaphoreType.DMA,
        ],
    )(_sc_body)
    out = run(xf, meta, jj_r, ng, gc)
    return out.reshape(B, MAX_PAD_LEN, D)
